# Initial kernel scaffold; baseline (speedup 1.0000x reference)
import jax
import jax.numpy as jnp
from jax.experimental import pallas as pl


def kernel(idx, embedding):
    raise NotImplementedError("write your pallas kernel here")


# speedup vs baseline: 1.8146x; 1.8146x over previous
import functools

import jax
import jax.numpy as jnp
from jax import lax
from jax.experimental import pallas as pl
from jax.experimental.pallas import tpu as pltpu
from jax.experimental.pallas import tpu_sc as plsc

_INFO = plsc.get_sparse_core_info()
_NC = _INFO.num_cores
_NS = _INFO.num_subcores
_NW = _NC * _NS


def _make_gather(n_tok: int, d: int, chunk: int):
    b_per_w = n_tok // _NW
    n_chunks = b_per_w // chunk
    mesh = plsc.VectorSubcoreMesh(core_axis_name="c", subcore_axis_name="s")

    @functools.partial(
        pl.kernel,
        mesh=mesh,
        out_type=jax.ShapeDtypeStruct((n_tok, d), jnp.float32),
        scratch_types=[
            pltpu.VMEM((n_chunks, chunk), jnp.int32),
            pltpu.VMEM((chunk, d), jnp.float32),
            pltpu.SemaphoreType.DMA,
        ],
    )
    def gather_kernel(table_hbm, idx_hbm, out_hbm, idx_v, buf, sem):
        wid = lax.axis_index("s") * _NC + lax.axis_index("c")
        base = wid * b_per_w
        pltpu.sync_copy(idx_hbm.at[wid], idx_v)

        def chunk_body(c, carry):
            pltpu.async_copy(table_hbm.at[idx_v.at[c]], buf, sem).wait()
            pltpu.sync_copy(buf, out_hbm.at[pl.ds(base + c * chunk, chunk)])
            return carry

        lax.fori_loop(0, n_chunks, chunk_body, 0)

    return gather_kernel


def kernel(idx, embedding):
    b, s = idx.shape
    v, d = embedding.shape
    n_tok = b * s
    chunk = 8
    idx32 = idx.reshape(_NW, (n_tok // _NW) // chunk, chunk).astype(jnp.int32)
    out = _make_gather(n_tok, d, chunk)(embedding, idx32)
    return out.reshape(b, s, d)


# speedup vs baseline: 1.9353x; 1.0665x over previous
import functools

import jax
import jax.numpy as jnp
from jax import lax
from jax.experimental import pallas as pl
from jax.experimental.pallas import tpu as pltpu
from jax.experimental.pallas import tpu_sc as plsc

_INFO = plsc.get_sparse_core_info()
_NC = _INFO.num_cores
_NS = _INFO.num_subcores
_NW = _NC * _NS


def _make_gather(n_tok: int, d: int, chunk: int):
    b_per_w = n_tok // _NW
    n_chunks = b_per_w // chunk
    assert n_chunks % 2 == 0
    mesh = plsc.VectorSubcoreMesh(core_axis_name="c", subcore_axis_name="s")

    @functools.partial(
        pl.kernel,
        mesh=mesh,
        out_type=jax.ShapeDtypeStruct((n_tok, d), jnp.float32),
        scratch_types=[
            pltpu.VMEM((n_chunks, chunk), jnp.int32),
            pltpu.VMEM((chunk, d), jnp.float32),
            pltpu.VMEM((chunk, d), jnp.float32),
            pltpu.SemaphoreType.DMA,
            pltpu.SemaphoreType.DMA,
            pltpu.SemaphoreType.DMA,
            pltpu.SemaphoreType.DMA,
        ],
    )
    def gather_kernel(table_hbm, idx_hbm, out_hbm, idx_v, buf0, buf1,
                      gsem0, gsem1, wsem0, wsem1):
        wid = lax.axis_index("s") * _NC + lax.axis_index("c")
        base = wid * b_per_w
        pltpu.sync_copy(idx_hbm.at[wid], idx_v)

        bufs = (buf0, buf1)
        gsems = (gsem0, gsem1)
        wsems = (wsem0, wsem1)

        def out_rows(g):
            return out_hbm.at[pl.ds(base + g * chunk, chunk)]

        pltpu.async_copy(table_hbm.at[idx_v.at[0]], buf0, gsem0)

        def pair_body(t, carry):
            for b in range(2):
                g = 2 * t + b
                pltpu.make_async_copy(
                    table_hbm.at[idx_v.at[g]], bufs[b], gsems[b]).wait()

                @pl.when(g >= 1)
                def _():
                    pltpu.make_async_copy(
                        bufs[1 - b], out_rows(g - 1), wsems[1 - b]).wait()

                @pl.when(g + 1 < n_chunks)
                def _():
                    pltpu.async_copy(
                        table_hbm.at[idx_v.at[g + 1]], bufs[1 - b],
                        gsems[1 - b])

                pltpu.async_copy(bufs[b], out_rows(g), wsems[b])
            return carry

        lax.fori_loop(0, n_chunks // 2, pair_body, 0)

        pltpu.make_async_copy(
            bufs[1], out_rows(n_chunks - 1), wsems[1]).wait()

    return gather_kernel


def kernel(idx, embedding):
    b, s = idx.shape
    v, d = embedding.shape
    n_tok = b * s
    chunk = 4
    idx32 = idx.reshape(_NW, (n_tok // _NW) // chunk, chunk).astype(jnp.int32)
    out = _make_gather(n_tok, d, chunk)(embedding, idx32)
    return out.reshape(b, s, d)


# speedup vs baseline: 1.9566x; 1.0110x over previous
import functools

import jax
import jax.numpy as jnp
from jax import lax
from jax.experimental import pallas as pl
from jax.experimental.pallas import tpu as pltpu
from jax.experimental.pallas import tpu_sc as plsc

_INFO = plsc.get_sparse_core_info()
_NC = _INFO.num_cores
_NS = _INFO.num_subcores
_NW = _NC * _NS

_NBUF = 4


def _make_gather(n_tok: int, d: int, chunk: int):
    b_per_w = n_tok // _NW
    n_chunks = b_per_w // chunk
    assert n_chunks % _NBUF == 0 and n_chunks >= 2 * _NBUF
    mesh = plsc.VectorSubcoreMesh(core_axis_name="c", subcore_axis_name="s")

    @functools.partial(
        pl.kernel,
        mesh=mesh,
        out_type=jax.ShapeDtypeStruct((n_tok, d), jnp.float32),
        scratch_types=[
            pltpu.VMEM((n_chunks, chunk), jnp.int32),
        ] + [pltpu.VMEM((chunk, d), jnp.float32)] * _NBUF
          + [pltpu.SemaphoreType.DMA] * (2 * _NBUF),
    )
    def gather_kernel(table_hbm, idx_hbm, out_hbm, idx_v, *rest):
        bufs = rest[:_NBUF]
        gsems = rest[_NBUF:2 * _NBUF]
        wsems = rest[2 * _NBUF:]

        wid = lax.axis_index("s") * _NC + lax.axis_index("c")
        base = wid * b_per_w
        pltpu.sync_copy(idx_hbm.at[wid], idx_v)

        def out_rows(g):
            return out_hbm.at[pl.ds(base + g * chunk, chunk)]

        def start_gather(g, b):
            pltpu.async_copy(table_hbm.at[idx_v.at[g]], bufs[b], gsems[b])

        for b in range(_NBUF - 1):
            start_gather(b, b)

        def ring_body(t, carry):
            for b in range(_NBUF):
                g = _NBUF * t + b
                bn = (b + _NBUF - 1) % _NBUF
                pltpu.make_async_copy(
                    table_hbm.at[idx_v.at[g]], bufs[b], gsems[b]).wait()

                @pl.when((g >= 1) & (g + _NBUF - 1 < n_chunks))
                def _():
                    pltpu.make_async_copy(
                        bufs[bn], out_rows(g - 1), wsems[bn]).wait()

                @pl.when(g + _NBUF - 1 < n_chunks)
                def _():
                    start_gather(g + _NBUF - 1, bn)

                pltpu.async_copy(bufs[b], out_rows(g), wsems[b])
            return carry

        lax.fori_loop(0, n_chunks // _NBUF, ring_body, 0)

        for b in range(_NBUF):
            g = n_chunks - _NBUF + b
            pltpu.make_async_copy(bufs[b], out_rows(g), wsems[b]).wait()

    return gather_kernel


def kernel(idx, embedding):
    b, s = idx.shape
    v, d = embedding.shape
    n_tok = b * s
    chunk = 2
    idx32 = idx.reshape(_NW, (n_tok // _NW) // chunk, chunk).astype(jnp.int32)
    out = _make_gather(n_tok, d, chunk)(embedding, idx32)
    return out.reshape(b, s, d)
